# baseline (device time: 13449 ns/iter reference)
import jax
import jax.numpy as jnp
from jax import lax
from jax.experimental import pallas as pl
from jax.experimental.pallas import tpu as pltpu

N_DEV = 4
N_TOK = 512
D_MODEL = 256
D_OUT = 512
N_EXP = 8
ROWS_PER = N_TOK // N_DEV

_SEND_ORDER = (2, 1, 3)


def kernel(x, router_W, route_idx, expert_W, shared_W):
    def body(
        x_hbm,
        router_ref,
        ridx_ref,
        expert_hbm,
        shared_hbm,
        out_ref,
        x_ref,
        expert_ref,
        shared_ref,
        partial_ref,
        comm_ref,
        x_sems,
        in_sems,
        send_sems,
        recv_sems,
    ):
        me = lax.axis_index("i")

        barrier_sem = pltpu.get_barrier_semaphore()
        for h in range(1, N_DEV):
            peer = lax.rem(me + h, N_DEV)
            pl.semaphore_signal(
                barrier_sem,
                inc=1,
                device_id=(peer,),
                device_id_type=pl.DeviceIdType.MESH,
            )

        w_dma = pltpu.make_async_copy(expert_hbm, expert_ref, in_sems.at[0])
        w_dma.start()
        x_dmas = []
        for k, h in enumerate((*_SEND_ORDER, 0)):
            row0 = lax.rem(me + h, N_DEV) * ROWS_PER
            dma = pltpu.make_async_copy(
                x_hbm.at[pl.ds(row0, ROWS_PER), :],
                x_ref.at[pl.ds(row0, ROWS_PER), :],
                x_sems.at[k],
            )
            dma.start()
            x_dmas.append(dma)
        s_dma = pltpu.make_async_copy(shared_hbm, shared_ref, in_sems.at[1])
        s_dma.start()

        w_dma.wait()
        w0 = expert_ref[0]
        w1 = expert_ref[1]

        def block_scales(row0):
            xb = x_ref[pl.ds(row0, ROWS_PER), :]
            scores = lax.dot_general(
                xb,
                router_ref[:, :],
                dimension_numbers=(((1,), (1,)), ((), ())),
                preferred_element_type=jnp.float32,
            )
            scores = scores - jnp.max(scores, axis=1, keepdims=True)
            e = jnp.exp(scores)
            probs = e / jnp.sum(e, axis=1, keepdims=True)
            ridx = ridx_ref[pl.ds(row0, ROWS_PER), :]
            col = lax.broadcasted_iota(jnp.int32, (ROWS_PER, N_EXP), 1)
            p = jnp.sum(jnp.where(col == ridx, probs, 0.0), axis=1, keepdims=True)
            s0 = p * (ridx == 2 * me).astype(jnp.float32)
            s1 = p * (ridx == 2 * me + 1).astype(jnp.float32)
            return xb.astype(jnp.bfloat16), s0, s1

        pl.semaphore_wait(barrier_sem, N_DEV - 1)

        sends = []
        for k, h in enumerate(_SEND_ORDER):
            tgt = lax.rem(me + h, N_DEV)
            x_dmas[k].wait()
            xb, s0, s1 = block_scales(tgt * ROWS_PER)
            y0 = jnp.dot(xb, w0, preferred_element_type=jnp.float32)
            y1 = jnp.dot(xb, w1, preferred_element_type=jnp.float32)
            partial_ref[h - 1] = (s0 * y0 + s1 * y1).astype(jnp.bfloat16)
            rdma = pltpu.make_async_remote_copy(
                src_ref=partial_ref.at[h - 1],
                dst_ref=comm_ref.at[h - 1],
                send_sem=send_sems.at[h - 1],
                recv_sem=recv_sems.at[h - 1],
                device_id=(tgt,),
                device_id_type=pl.DeviceIdType.MESH,
            )
            rdma.start()
            sends.append(rdma)

        x_dmas[3].wait()
        xb, s0, s1 = block_scales(me * ROWS_PER)
        y0 = jnp.dot(xb, w0, preferred_element_type=jnp.float32)
        y1 = jnp.dot(xb, w1, preferred_element_type=jnp.float32)
        s_dma.wait()
        shared = jnp.dot(
            xb, shared_ref[:, :], preferred_element_type=jnp.float32
        )
        acc = shared + s0 * y0 + s1 * y1

        for h, rdma in zip(_SEND_ORDER, sends):
            rdma.wait()
            acc = acc + comm_ref[h - 1].astype(jnp.float32)

        out_ref[:, :] = acc

    return pl.pallas_call(
        body,
        out_shape=jax.ShapeDtypeStruct((ROWS_PER, D_OUT), jnp.float32),
        in_specs=[
            pl.BlockSpec(memory_space=pl.ANY),
            pl.BlockSpec(memory_space=pltpu.VMEM),
            pl.BlockSpec(memory_space=pltpu.VMEM),
            pl.BlockSpec(memory_space=pl.ANY),
            pl.BlockSpec(memory_space=pl.ANY),
        ],
        out_specs=pl.BlockSpec(memory_space=pltpu.VMEM),
        scratch_shapes=[
            pltpu.VMEM((N_TOK, D_MODEL), jnp.float32),
            pltpu.VMEM((2, D_MODEL, D_OUT), jnp.bfloat16),
            pltpu.VMEM((D_MODEL, D_OUT), jnp.bfloat16),
            pltpu.VMEM((N_DEV - 1, ROWS_PER, D_OUT), jnp.bfloat16),
            pltpu.VMEM((N_DEV - 1, ROWS_PER, D_OUT), jnp.bfloat16),
            pltpu.SemaphoreType.DMA((N_DEV,)),
            pltpu.SemaphoreType.DMA((2,)),
            pltpu.SemaphoreType.DMA((N_DEV - 1,)),
            pltpu.SemaphoreType.DMA((N_DEV - 1,)),
        ],
        compiler_params=pltpu.CompilerParams(collective_id=0),
    )(
        pltpu.with_memory_space_constraint(x, pltpu.MemorySpace.HBM),
        router_W.T,
        route_idx,
        pltpu.with_memory_space_constraint(
            expert_W.astype(jnp.bfloat16), pltpu.MemorySpace.HBM
        ),
        pltpu.with_memory_space_constraint(
            shared_W.astype(jnp.bfloat16), pltpu.MemorySpace.HBM
        ),
    )


# device time: 11031 ns/iter; 1.2192x vs baseline; 1.2192x over previous
import jax
import jax.numpy as jnp
from jax import lax
from jax.experimental import pallas as pl
from jax.experimental.pallas import tpu as pltpu

N_DEV = 4
N_TOK = 512
D_MODEL = 256
D_OUT = 512
N_EXP = 8
ROWS_PER = N_TOK // N_DEV

_SEND_ORDER = (2, 1, 3)


def kernel(x, router_W, route_idx, expert_W, shared_W):
    def body(
        x_hbm,
        router_ref,
        ridx_ref,
        expert_hbm,
        shared_hbm,
        out_ref,
        x_ref,
        expert_ref,
        shared_ref,
        partial_ref,
        comm_ref,
        x_sems,
        in_sems,
        send_sems,
        recv_sems,
    ):
        me = lax.axis_index("i")

        barrier_sem = pltpu.get_barrier_semaphore()
        for h in range(1, N_DEV):
            peer = lax.rem(me + h, N_DEV)
            pl.semaphore_signal(
                barrier_sem,
                inc=1,
                device_id=(peer,),
                device_id_type=pl.DeviceIdType.MESH,
            )

        w_dma = pltpu.make_async_copy(expert_hbm, expert_ref, in_sems.at[0])
        w_dma.start()
        x_dmas = []
        for k, h in enumerate((*_SEND_ORDER, 0)):
            row0 = lax.rem(me + h, N_DEV) * ROWS_PER
            dma = pltpu.make_async_copy(
                x_hbm.at[pl.ds(row0, ROWS_PER), :],
                x_ref.at[pl.ds(row0, ROWS_PER), :],
                x_sems.at[k],
            )
            dma.start()
            x_dmas.append(dma)
        s_dma = pltpu.make_async_copy(shared_hbm, shared_ref, in_sems.at[1])
        s_dma.start()

        w_dma.wait()
        w0 = expert_ref[0].astype(jnp.bfloat16)
        w1 = expert_ref[1].astype(jnp.bfloat16)

        def block_scales(row0):
            xb = x_ref[pl.ds(row0, ROWS_PER), :]
            scores = lax.dot_general(
                xb,
                router_ref[:, :],
                dimension_numbers=(((1,), (1,)), ((), ())),
                preferred_element_type=jnp.float32,
            )
            scores = scores - jnp.max(scores, axis=1, keepdims=True)
            e = jnp.exp(scores)
            probs = e / jnp.sum(e, axis=1, keepdims=True)
            ridx = ridx_ref[pl.ds(row0, ROWS_PER), :]
            col = lax.broadcasted_iota(jnp.int32, (ROWS_PER, N_EXP), 1)
            p = jnp.sum(jnp.where(col == ridx, probs, 0.0), axis=1, keepdims=True)
            s0 = p * (ridx == 2 * me).astype(jnp.float32)
            s1 = p * (ridx == 2 * me + 1).astype(jnp.float32)
            return xb.astype(jnp.bfloat16), s0, s1

        pl.semaphore_wait(barrier_sem, N_DEV - 1)

        sends = []
        for k, h in enumerate(_SEND_ORDER):
            tgt = lax.rem(me + h, N_DEV)
            x_dmas[k].wait()
            xb, s0, s1 = block_scales(tgt * ROWS_PER)
            y0 = jnp.dot(xb, w0, preferred_element_type=jnp.float32)
            y1 = jnp.dot(xb, w1, preferred_element_type=jnp.float32)
            partial_ref[h - 1] = (s0 * y0 + s1 * y1).astype(jnp.bfloat16)
            rdma = pltpu.make_async_remote_copy(
                src_ref=partial_ref.at[h - 1],
                dst_ref=comm_ref.at[h - 1],
                send_sem=send_sems.at[h - 1],
                recv_sem=recv_sems.at[h - 1],
                device_id=(tgt,),
                device_id_type=pl.DeviceIdType.MESH,
            )
            rdma.start()
            sends.append(rdma)

        x_dmas[3].wait()
        xb, s0, s1 = block_scales(me * ROWS_PER)
        y0 = jnp.dot(xb, w0, preferred_element_type=jnp.float32)
        y1 = jnp.dot(xb, w1, preferred_element_type=jnp.float32)
        s_dma.wait()
        shared = jnp.dot(
            xb, shared_ref[:, :].astype(jnp.bfloat16), preferred_element_type=jnp.float32
        )
        acc = shared + s0 * y0 + s1 * y1

        for h, rdma in zip(_SEND_ORDER, sends):
            rdma.wait()
            acc = acc + comm_ref[h - 1].astype(jnp.float32)

        out_ref[:, :] = acc

    return pl.pallas_call(
        body,
        out_shape=jax.ShapeDtypeStruct((ROWS_PER, D_OUT), jnp.float32),
        in_specs=[
            pl.BlockSpec(memory_space=pl.ANY),
            pl.BlockSpec(memory_space=pltpu.VMEM),
            pl.BlockSpec(memory_space=pltpu.VMEM),
            pl.BlockSpec(memory_space=pl.ANY),
            pl.BlockSpec(memory_space=pl.ANY),
        ],
        out_specs=pl.BlockSpec(memory_space=pltpu.VMEM),
        scratch_shapes=[
            pltpu.VMEM((N_TOK, D_MODEL), jnp.float32),
            pltpu.VMEM((2, D_MODEL, D_OUT), jnp.float32),
            pltpu.VMEM((D_MODEL, D_OUT), jnp.float32),
            pltpu.VMEM((N_DEV - 1, ROWS_PER, D_OUT), jnp.bfloat16),
            pltpu.VMEM((N_DEV - 1, ROWS_PER, D_OUT), jnp.bfloat16),
            pltpu.SemaphoreType.DMA((N_DEV,)),
            pltpu.SemaphoreType.DMA((2,)),
            pltpu.SemaphoreType.DMA((N_DEV - 1,)),
            pltpu.SemaphoreType.DMA((N_DEV - 1,)),
        ],
        compiler_params=pltpu.CompilerParams(collective_id=0),
    )(
        pltpu.with_memory_space_constraint(x, pltpu.MemorySpace.HBM),
        router_W.T,
        route_idx,
        pltpu.with_memory_space_constraint(expert_W, pltpu.MemorySpace.HBM),
        pltpu.with_memory_space_constraint(shared_W, pltpu.MemorySpace.HBM),
    )
